# baseline (speedup 1.0000x reference)
import functools

import jax
import jax.numpy as jnp
from jax import lax
from jax.experimental import pallas as pl
from jax.experimental.pallas import tpu as pltpu
from jax.experimental.pallas import tpu_sc as plsc


def _density_mv_body(w1_ref, b1_ref, t_ref, x_ref):
    t = t_ref[0]
    w1 = w1_ref[...]
    acc = lax.dot_general(w1, t, (((1,), (0,)), ((), ())),
                          preferred_element_type=jnp.float32,
                          precision=lax.Precision.DEFAULT)
    x_ref[0] = acc + b1_ref[0]


def _density_mv(tflat, w1, b1):
    B, C, HW = tflat.shape
    TILE = 1024
    grid = (B, HW // TILE)
    return pl.pallas_call(
        _density_mv_body,
        grid=grid,
        in_specs=[
            pl.BlockSpec((1, C), lambda b, t: (0, 0)),
            pl.BlockSpec(memory_space=pltpu.SMEM),
            pl.BlockSpec((1, C, TILE), lambda b, t: (b, 0, t)),
        ],
        out_specs=pl.BlockSpec((1, 1, TILE), lambda b, t: (b, 0, t)),
        out_shape=jax.ShapeDtypeStruct((B, 1, HW), jnp.float32),
    )(w1, b1, tflat)


def _shift2d(x, dy, dx):
    H, W = x.shape
    if dy > 0:
        x = jnp.concatenate([x[dy:, :], jnp.zeros((dy, W), x.dtype)], axis=0)
    elif dy < 0:
        x = jnp.concatenate([jnp.zeros((-dy, W), x.dtype), x[:dy, :]], axis=0)
    if dx > 0:
        x = jnp.concatenate([x[:, dx:], jnp.zeros((H, dx), x.dtype)], axis=1)
    elif dx < 0:
        x = jnp.concatenate([jnp.zeros((H, -dx), x.dtype), x[:, :dx]], axis=1)
    return x


def _conv_body(w2_ref, b2_ref, x_ref, d_ref):
    x = x_ref[0]
    xb = x.astype(jnp.bfloat16).astype(jnp.float32)
    acc = jnp.zeros_like(x)
    for i in range(3):
        for j in range(3):
            w = w2_ref[i * 3 + j].astype(jnp.bfloat16).astype(jnp.float32)
            acc = acc + w * _shift2d(xb, 2 * (i - 1), 2 * (j - 1))
    d_ref[0] = jnp.maximum(acc + b2_ref[0] + x, 0.0)


def _conv_relu(x, w2, b2):
    B, H, W = x.shape
    return pl.pallas_call(
        _conv_body,
        grid=(B,),
        in_specs=[
            pl.BlockSpec(memory_space=pltpu.SMEM),
            pl.BlockSpec(memory_space=pltpu.SMEM),
            pl.BlockSpec((1, H, W), lambda b: (b, 0, 0)),
        ],
        out_specs=pl.BlockSpec((1, H, W), lambda b: (b, 0, 0)),
        out_shape=jax.ShapeDtypeStruct((B, H, W), jnp.float32),
    )(w2, b2, x)


def _xor_shuffle(arr, pos, d):
    if d < 128:
        ax, sz, ds = 2, arr.shape[2], d
    else:
        ax, sz, ds = 1, arr.shape[1], d // 128
    r_p = pltpu.roll(arr, ds, axis=ax)
    r_m = pltpu.roll(arr, sz - ds, axis=ax)
    bit = (pos & d) != 0
    return jnp.where(bit, r_p, r_m)


def _sort_body(g_ref, d_ref, te_ref, wt_ref, bt_ref,
               topk_ref, tflat_ref, mask_ref, tadd_ref):
    B, S, L = d_ref.shape
    N = S * L
    K = N // 2
    v0 = d_ref[...]
    s_iota = lax.broadcasted_iota(jnp.int32, (B, S, L), 1)
    l_iota = lax.broadcasted_iota(jnp.int32, (B, S, L), 2)
    pos = s_iota * L + l_iota

    v = v0
    idx = pos
    for m in range(1, 13):
        kk = 1 << m
        for jb in range(m - 1, -1, -1):
            d = 1 << jb
            pv = _xor_shuffle(v, pos, d)
            pidx = _xor_shuffle(idx, pos, d)
            g = (v > pv) | ((v == pv) & (idx < pidx))
            is_upper = (pos & d) != 0
            is_asc = (pos & kk) != 0
            keep = g ^ is_upper ^ is_asc
            v = jnp.where(keep, v, pv)
            idx = jnp.where(keep, idx, pidx)

    topk = idx[:, :S // 2, :]
    topk_ref[...] = topk
    b_iota = lax.broadcasted_iota(jnp.int32, (B, S // 2, L), 0)
    tflat_ref[...] = topk + b_iota * N

    t = v[:, (S // 2 - 1):(S // 2), (L - 1):L]
    gt = (v0 > t)
    eq = (v0 == t)
    n_gt = jnp.sum(gt.astype(jnp.float32), axis=(1, 2), keepdims=True)
    c = eq.astype(jnp.float32)
    sh = 1
    while sh < L:
        rolled = pltpu.roll(c, sh, axis=2)
        c = c + jnp.where(l_iota >= sh, rolled, 0.0)
        sh *= 2
    row_tot = c[:, :, (L - 1):L]
    inc = row_tot
    sh = 1
    while sh < S:
        rolled = pltpu.roll(inc, sh, axis=1)
        s1 = lax.broadcasted_iota(jnp.int32, (B, S, 1), 1)
        inc = inc + jnp.where(s1 >= sh, rolled, 0.0)
        sh *= 2
    excl = inc - row_tot
    cs = c + excl
    quota = jnp.float32(K) - n_gt
    sel = gt | (eq & (cs <= quota))
    mask_ref[...] = sel.astype(jnp.float32)

    te = te_ref[...]
    wt = wt_ref[...]
    tf = lax.dot_general(te, wt, (((1,), (1,)), ((), ())),
                         preferred_element_type=jnp.float32) + bt_ref[...]
    tf = jnp.maximum(tf, 0.0) * 0.1
    mu = jnp.mean(tf, axis=1, keepdims=True)
    var = jnp.mean((tf - mu) * (tf - mu), axis=1, keepdims=True)
    tfn = (tf - mu) * lax.rsqrt(var + 1e-5)
    gate = 1.0 / (1.0 + jnp.exp(-g_ref[0]))
    tadd_ref[...] = tfn * gate


def _sort_topk(d3, text_emb, wt, bt, gate_param):
    B, S, L = d3.shape
    E, T = wt.shape
    out_shapes = (
        jax.ShapeDtypeStruct((B, S // 2, L), jnp.int32),
        jax.ShapeDtypeStruct((B, S // 2, L), jnp.int32),
        jax.ShapeDtypeStruct((B, S, L), jnp.float32),
        jax.ShapeDtypeStruct((B, E), jnp.float32),
    )
    return pl.pallas_call(
        _sort_body,
        in_specs=[
            pl.BlockSpec(memory_space=pltpu.SMEM),
            pl.BlockSpec((B, S, L), lambda: (0, 0, 0)),
            pl.BlockSpec((B, T), lambda: (0, 0)),
            pl.BlockSpec((E, T), lambda: (0, 0)),
            pl.BlockSpec((1, E), lambda: (0, 0)),
        ],
        out_specs=(
            pl.BlockSpec((B, S // 2, L), lambda: (0, 0, 0)),
            pl.BlockSpec((B, S // 2, L), lambda: (0, 0, 0)),
            pl.BlockSpec((B, S, L), lambda: (0, 0, 0)),
            pl.BlockSpec((B, E), lambda: (0, 0)),
        ),
        out_shape=out_shapes,
    )(gate_param, d3, text_emb, wt, bt)


def _fused_body(t_ref, m_ref, ws_ref, bs_ref, wo_ref, bo_ref, ta_ref,
                out_ref, tmp_ref):
    x = t_ref[0]
    ws = ws_ref[...]
    wo = wo_ref[...]
    mid = lax.dot_general(ws, x, (((1,), (0,)), ((), ())),
                          preferred_element_type=jnp.float32) + bs_ref[...]
    E = mid.shape[0]
    mu = jnp.mean(mid, axis=0, keepdims=True)
    var = jnp.mean((mid - mu) * (mid - mu), axis=0, keepdims=True)
    normed = (mid - mu) * lax.rsqrt(var + 1e-5)
    ta_col = jnp.transpose(ta_ref[0])
    fpre = normed + ta_col
    outc = lax.dot_general(wo, fpre, (((1,), (0,)), ((), ())),
                           preferred_element_type=jnp.float32) + bo_ref[...]
    s_in = jnp.sqrt(jnp.sum(x * x, axis=0, keepdims=True))
    s_out = jnp.sqrt(jnp.sum(outc * outc, axis=0, keepdims=True))
    scale = s_in / jnp.maximum(s_out, 1e-12)
    outs = outc * scale
    sel = m_ref[0] > 0.0
    out_ref[0] = jnp.where(sel, outs, x)
    tmp_ref[0] = jnp.transpose(outs)


def _fused(tflat, mask, ws, bs_col, wo, bo_col, text_add):
    B, C, HW = tflat.shape
    E = ws.shape[0]
    TILE = 1024
    grid = (B, HW // TILE)
    return pl.pallas_call(
        _fused_body,
        grid=grid,
        in_specs=[
            pl.BlockSpec((1, C, TILE), lambda b, t: (b, 0, t)),
            pl.BlockSpec((1, 1, TILE), lambda b, t: (b, 0, t)),
            pl.BlockSpec((E, C), lambda b, t: (0, 0)),
            pl.BlockSpec((E, 1), lambda b, t: (0, 0)),
            pl.BlockSpec((C, E), lambda b, t: (0, 0)),
            pl.BlockSpec((C, 1), lambda b, t: (0, 0)),
            pl.BlockSpec((1, 1, E), lambda b, t: (b, 0, 0)),
        ],
        out_specs=(
            pl.BlockSpec((1, C, TILE), lambda b, t: (b, 0, t)),
            pl.BlockSpec((1, TILE, C), lambda b, t: (b, t, 0)),
        ),
        out_shape=(
            jax.ShapeDtypeStruct((B, C, HW), jnp.float32),
            jax.ShapeDtypeStruct((B, HW, C), jnp.float32),
        ),
    )(tflat, mask, ws, bs_col, wo, bo_col, text_add)


_NC, _NS = 2, 16
_NW = _NC * _NS


def _sc_gather(tmp_rows, idx_flat, C):
    (NR,) = idx_flat.shape
    R = NR // _NW
    CH = 64
    NCHUNK = R // CH
    mesh = plsc.VectorSubcoreMesh(core_axis_name="c", subcore_axis_name="s")

    @functools.partial(
        pl.kernel, mesh=mesh,
        out_type=jax.ShapeDtypeStruct((NR, C), jnp.float32),
        scratch_types=[
            pltpu.VMEM((R,), jnp.int32),
            pltpu.VMEM((CH, C), jnp.float32),
            pltpu.VMEM((CH, C), jnp.float32),
            pltpu.SemaphoreType.DMA,
            pltpu.SemaphoreType.DMA,
            pltpu.SemaphoreType.DMA,
            pltpu.SemaphoreType.DMA,
        ],
    )
    def k(tmp_hbm, idx_hbm, out_hbm, idx_v, rows0, rows1, sg0, sg1, sw0, sw1):
        wid = lax.axis_index("s") * _NC + lax.axis_index("c")
        base = wid * R
        pltpu.sync_copy(idx_hbm.at[pl.ds(base, R)], idx_v)
        rows = (rows0, rows1)
        sg = (sg0, sg1)
        sw = (sw0, sw1)
        gathers = [None] * NCHUNK
        writes = [None] * NCHUNK
        gathers[0] = pltpu.async_copy(
            tmp_hbm.at[idx_v.at[pl.ds(0, CH)]], rows[0], sg[0])
        for i in range(NCHUNK):
            b = i % 2
            if i + 1 < NCHUNK:
                b2 = (i + 1) % 2
                if i >= 1:
                    writes[i - 1].wait()
                gathers[i + 1] = pltpu.async_copy(
                    tmp_hbm.at[idx_v.at[pl.ds((i + 1) * CH, CH)]],
                    rows[b2], sg[b2])
            gathers[i].wait()
            writes[i] = pltpu.async_copy(
                rows[b], out_hbm.at[pl.ds(base + i * CH, CH)], sw[b])
        writes[NCHUNK - 2].wait()
        writes[NCHUNK - 1].wait()

    return k(tmp_rows, idx_flat)


def kernel(tensor, text_emb, W1, b1, W2, b2, Ws, bs, Wt, bt, Wo, bo, gate_param):
    B, C, H, Wd = tensor.shape
    HW = H * Wd
    K = HW // 2
    E = Ws.shape[0]

    tflat = tensor.reshape(B, C, HW)
    x = _density_mv(tflat, W1.reshape(1, C), b1)
    dens = _conv_relu(x.reshape(B, H, Wd), W2.reshape(9), b2)
    d3 = dens.reshape(B, HW // 128, 128)
    topk3, tflat3, mask3, text_add = _sort_topk(
        d3, text_emb, Wt, bt.reshape(1, E), gate_param)
    topk_idx = topk3.reshape(B, K)
    idx_flat = tflat3.reshape(B * K)
    mask = mask3.reshape(B, 1, HW)
    out, tmp = _fused(tflat, mask, Ws, bs.reshape(E, 1), Wo, bo.reshape(C, 1),
                      text_add.reshape(B, 1, E))
    fused_sparse = _sc_gather(tmp.reshape(B * HW, C), idx_flat, C)
    return (out.reshape(B, C, H, Wd), dens.reshape(B, 1, H, Wd), topk_idx,
            fused_sparse.reshape(B, K, C))


# speedup vs baseline: 1.0793x; 1.0793x over previous
import functools

import jax
import jax.numpy as jnp
from jax import lax
from jax.experimental import pallas as pl
from jax.experimental.pallas import tpu as pltpu
from jax.experimental.pallas import tpu_sc as plsc


def _density_mv_body(w1_ref, b1_ref, t_ref, x_ref):
    t = t_ref[0]
    w1 = w1_ref[...]
    acc = lax.dot_general(w1, t, (((1,), (0,)), ((), ())),
                          preferred_element_type=jnp.float32,
                          precision=lax.Precision.DEFAULT)
    x_ref[0] = acc + b1_ref[0]


def _density_mv(tflat, w1, b1):
    B, C, HW = tflat.shape
    TILE = 4096
    grid = (B, HW // TILE)
    return pl.pallas_call(
        _density_mv_body,
        grid=grid,
        in_specs=[
            pl.BlockSpec((1, C), lambda b, t: (0, 0)),
            pl.BlockSpec(memory_space=pltpu.SMEM),
            pl.BlockSpec((1, C, TILE), lambda b, t: (b, 0, t)),
        ],
        out_specs=pl.BlockSpec((1, 1, TILE), lambda b, t: (b, 0, t)),
        out_shape=jax.ShapeDtypeStruct((B, 1, HW), jnp.float32),
    )(w1, b1, tflat)


def _shift2d(x, dy, dx):
    H, W = x.shape
    if dy > 0:
        x = jnp.concatenate([x[dy:, :], jnp.zeros((dy, W), x.dtype)], axis=0)
    elif dy < 0:
        x = jnp.concatenate([jnp.zeros((-dy, W), x.dtype), x[:dy, :]], axis=0)
    if dx > 0:
        x = jnp.concatenate([x[:, dx:], jnp.zeros((H, dx), x.dtype)], axis=1)
    elif dx < 0:
        x = jnp.concatenate([jnp.zeros((H, -dx), x.dtype), x[:, :dx]], axis=1)
    return x


def _conv_body(w2_ref, b2_ref, x_ref, d_ref):
    x = x_ref[0]
    xb = x.astype(jnp.bfloat16).astype(jnp.float32)
    acc = jnp.zeros_like(x)
    for i in range(3):
        for j in range(3):
            w = w2_ref[i * 3 + j].astype(jnp.bfloat16).astype(jnp.float32)
            acc = acc + w * _shift2d(xb, 2 * (i - 1), 2 * (j - 1))
    d_ref[0] = jnp.maximum(acc + b2_ref[0] + x, 0.0)


def _conv_relu(x, w2, b2):
    B, H, W = x.shape
    return pl.pallas_call(
        _conv_body,
        grid=(B,),
        in_specs=[
            pl.BlockSpec(memory_space=pltpu.SMEM),
            pl.BlockSpec(memory_space=pltpu.SMEM),
            pl.BlockSpec((1, H, W), lambda b: (b, 0, 0)),
        ],
        out_specs=pl.BlockSpec((1, H, W), lambda b: (b, 0, 0)),
        out_shape=jax.ShapeDtypeStruct((B, H, W), jnp.float32),
    )(w2, b2, x)


def _xor_shuffle(arr, pos, d):
    if d < 128:
        ax, sz, ds = 2, arr.shape[2], d
    else:
        ax, sz, ds = 1, arr.shape[1], d // 128
    r_p = pltpu.roll(arr, ds, axis=ax)
    r_m = pltpu.roll(arr, sz - ds, axis=ax)
    bit = (pos & d) != 0
    return jnp.where(bit, r_p, r_m)


def _sort_body(g_ref, d_ref, te_ref, wt_ref, bt_ref,
               topk_ref, tflat_ref, mask_ref, tadd_ref):
    B, S, L = d_ref.shape
    N = S * L
    K = N // 2
    v0 = d_ref[...]
    s_iota = lax.broadcasted_iota(jnp.int32, (B, S, L), 1)
    l_iota = lax.broadcasted_iota(jnp.int32, (B, S, L), 2)
    pos = s_iota * L + l_iota

    v = v0
    idx = pos
    for m in range(1, 13):
        kk = 1 << m
        for jb in range(m - 1, -1, -1):
            d = 1 << jb
            pv = _xor_shuffle(v, pos, d)
            pidx = _xor_shuffle(idx, pos, d)
            g = (v > pv) | ((v == pv) & (idx < pidx))
            is_upper = (pos & d) != 0
            is_asc = (pos & kk) != 0
            keep = g ^ is_upper ^ is_asc
            v = jnp.where(keep, v, pv)
            idx = jnp.where(keep, idx, pidx)

    topk = idx[:, :S // 2, :]
    topk_ref[...] = topk
    b_iota = lax.broadcasted_iota(jnp.int32, (B, S // 2, L), 0)
    tflat_ref[...] = topk + b_iota * N

    t = v[:, (S // 2 - 1):(S // 2), (L - 1):L]
    gt = (v0 > t)
    eq = (v0 == t)
    n_gt = jnp.sum(gt.astype(jnp.float32), axis=(1, 2), keepdims=True)
    c = eq.astype(jnp.float32)
    sh = 1
    while sh < L:
        rolled = pltpu.roll(c, sh, axis=2)
        c = c + jnp.where(l_iota >= sh, rolled, 0.0)
        sh *= 2
    row_tot = c[:, :, (L - 1):L]
    inc = row_tot
    sh = 1
    while sh < S:
        rolled = pltpu.roll(inc, sh, axis=1)
        s1 = lax.broadcasted_iota(jnp.int32, (B, S, 1), 1)
        inc = inc + jnp.where(s1 >= sh, rolled, 0.0)
        sh *= 2
    excl = inc - row_tot
    cs = c + excl
    quota = jnp.float32(K) - n_gt
    sel = gt | (eq & (cs <= quota))
    mask_ref[...] = sel.astype(jnp.float32)

    te = te_ref[...]
    wt = wt_ref[...]
    tf = lax.dot_general(te, wt, (((1,), (1,)), ((), ())),
                         preferred_element_type=jnp.float32) + bt_ref[...]
    tf = jnp.maximum(tf, 0.0) * 0.1
    mu = jnp.mean(tf, axis=1, keepdims=True)
    var = jnp.mean((tf - mu) * (tf - mu), axis=1, keepdims=True)
    tfn = (tf - mu) * lax.rsqrt(var + 1e-5)
    gate = 1.0 / (1.0 + jnp.exp(-g_ref[0]))
    tadd_ref[...] = tfn * gate


def _sort_topk(d3, text_emb, wt, bt, gate_param):
    B, S, L = d3.shape
    E, T = wt.shape
    out_shapes = (
        jax.ShapeDtypeStruct((B, S // 2, L), jnp.int32),
        jax.ShapeDtypeStruct((B, S // 2, L), jnp.int32),
        jax.ShapeDtypeStruct((B, S, L), jnp.float32),
        jax.ShapeDtypeStruct((B, E), jnp.float32),
    )
    return pl.pallas_call(
        _sort_body,
        in_specs=[
            pl.BlockSpec(memory_space=pltpu.SMEM),
            pl.BlockSpec((B, S, L), lambda: (0, 0, 0)),
            pl.BlockSpec((B, T), lambda: (0, 0)),
            pl.BlockSpec((E, T), lambda: (0, 0)),
            pl.BlockSpec((1, E), lambda: (0, 0)),
        ],
        out_specs=(
            pl.BlockSpec((B, S // 2, L), lambda: (0, 0, 0)),
            pl.BlockSpec((B, S // 2, L), lambda: (0, 0, 0)),
            pl.BlockSpec((B, S, L), lambda: (0, 0, 0)),
            pl.BlockSpec((B, E), lambda: (0, 0)),
        ),
        out_shape=out_shapes,
    )(gate_param, d3, text_emb, wt, bt)


def _fused_body(t_ref, m_ref, ws_ref, bs_ref, wo_ref, bo_ref, ta_ref,
                out_ref, tmp_ref):
    x = t_ref[0]
    ws = ws_ref[...]
    wo = wo_ref[...]
    mid = lax.dot_general(ws, x, (((1,), (0,)), ((), ())),
                          preferred_element_type=jnp.float32) + bs_ref[...]
    E = mid.shape[0]
    mu = jnp.mean(mid, axis=0, keepdims=True)
    var = jnp.mean((mid - mu) * (mid - mu), axis=0, keepdims=True)
    normed = (mid - mu) * lax.rsqrt(var + 1e-5)
    ta_col = jnp.transpose(ta_ref[0])
    fpre = normed + ta_col
    outc = lax.dot_general(wo, fpre, (((1,), (0,)), ((), ())),
                           preferred_element_type=jnp.float32) + bo_ref[...]
    s_in = jnp.sqrt(jnp.sum(x * x, axis=0, keepdims=True))
    s_out = jnp.sqrt(jnp.sum(outc * outc, axis=0, keepdims=True))
    scale = s_in / jnp.maximum(s_out, 1e-12)
    outs = outc * scale
    sel = m_ref[0] > 0.0
    out_ref[0] = jnp.where(sel, outs, x)
    tmp_ref[0] = jnp.transpose(outs)


def _fused(tflat, mask, ws, bs_col, wo, bo_col, text_add):
    B, C, HW = tflat.shape
    E = ws.shape[0]
    TILE = 2048
    grid = (B, HW // TILE)
    return pl.pallas_call(
        _fused_body,
        grid=grid,
        in_specs=[
            pl.BlockSpec((1, C, TILE), lambda b, t: (b, 0, t)),
            pl.BlockSpec((1, 1, TILE), lambda b, t: (b, 0, t)),
            pl.BlockSpec((E, C), lambda b, t: (0, 0)),
            pl.BlockSpec((E, 1), lambda b, t: (0, 0)),
            pl.BlockSpec((C, E), lambda b, t: (0, 0)),
            pl.BlockSpec((C, 1), lambda b, t: (0, 0)),
            pl.BlockSpec((1, 1, E), lambda b, t: (b, 0, 0)),
        ],
        out_specs=(
            pl.BlockSpec((1, C, TILE), lambda b, t: (b, 0, t)),
            pl.BlockSpec((1, TILE, C), lambda b, t: (b, t, 0)),
        ),
        out_shape=(
            jax.ShapeDtypeStruct((B, C, HW), jnp.float32),
            jax.ShapeDtypeStruct((B, HW, C), jnp.float32),
        ),
    )(tflat, mask, ws, bs_col, wo, bo_col, text_add)


_NC, _NS = 2, 16
_NW = _NC * _NS


def _sc_gather(tmp_rows, idx_flat, C):
    (NR,) = idx_flat.shape
    R = NR // _NW
    CH = 64
    NCHUNK = R // CH
    mesh = plsc.VectorSubcoreMesh(core_axis_name="c", subcore_axis_name="s")

    @functools.partial(
        pl.kernel, mesh=mesh,
        out_type=jax.ShapeDtypeStruct((NR, C), jnp.float32),
        scratch_types=[
            pltpu.VMEM((R,), jnp.int32),
            pltpu.VMEM((CH, C), jnp.float32),
            pltpu.VMEM((CH, C), jnp.float32),
            pltpu.SemaphoreType.DMA,
            pltpu.SemaphoreType.DMA,
            pltpu.SemaphoreType.DMA,
            pltpu.SemaphoreType.DMA,
        ],
    )
    def k(tmp_hbm, idx_hbm, out_hbm, idx_v, rows0, rows1, sg0, sg1, sw0, sw1):
        wid = lax.axis_index("s") * _NC + lax.axis_index("c")
        base = wid * R
        pltpu.sync_copy(idx_hbm.at[pl.ds(base, R)], idx_v)
        rows = (rows0, rows1)
        sg = (sg0, sg1)
        sw = (sw0, sw1)
        gathers = [None] * NCHUNK
        writes = [None] * NCHUNK
        gathers[0] = pltpu.async_copy(
            tmp_hbm.at[idx_v.at[pl.ds(0, CH)]], rows[0], sg[0])
        for i in range(NCHUNK):
            b = i % 2
            if i + 1 < NCHUNK:
                b2 = (i + 1) % 2
                if i >= 1:
                    writes[i - 1].wait()
                gathers[i + 1] = pltpu.async_copy(
                    tmp_hbm.at[idx_v.at[pl.ds((i + 1) * CH, CH)]],
                    rows[b2], sg[b2])
            gathers[i].wait()
            writes[i] = pltpu.async_copy(
                rows[b], out_hbm.at[pl.ds(base + i * CH, CH)], sw[b])
        writes[NCHUNK - 2].wait()
        writes[NCHUNK - 1].wait()

    return k(tmp_rows, idx_flat)


def kernel(tensor, text_emb, W1, b1, W2, b2, Ws, bs, Wt, bt, Wo, bo, gate_param):
    B, C, H, Wd = tensor.shape
    HW = H * Wd
    K = HW // 2
    E = Ws.shape[0]

    tflat = tensor.reshape(B, C, HW)
    x = _density_mv(tflat, W1.reshape(1, C), b1)
    dens = _conv_relu(x.reshape(B, H, Wd), W2.reshape(9), b2)
    d3 = dens.reshape(B, HW // 128, 128)
    topk3, tflat3, mask3, text_add = _sort_topk(
        d3, text_emb, Wt, bt.reshape(1, E), gate_param)
    topk_idx = topk3.reshape(B, K)
    idx_flat = tflat3.reshape(B * K)
    mask = mask3.reshape(B, 1, HW)
    out, tmp = _fused(tflat, mask, Ws, bs.reshape(E, 1), Wo, bo.reshape(C, 1),
                      text_add.reshape(B, 1, E))
    fused_sparse = _sc_gather(tmp.reshape(B * HW, C), idx_flat, C)
    return (out.reshape(B, C, H, Wd), dens.reshape(B, 1, H, Wd), topk_idx,
            fused_sparse.reshape(B, K, C))
